# Initial kernel scaffold; baseline (speedup 1.0000x reference)
import jax
import jax.numpy as jnp
from jax.experimental import pallas as pl


def kernel(x, Wq, Wk):
    raise NotImplementedError("write your pallas kernel here")


# speedup vs baseline: 3.2132x; 3.2132x over previous
import math

import jax
import jax.numpy as jnp
from jax.experimental import pallas as pl
from jax.experimental.pallas import tpu as pltpu

_C = 768
_T = 2048
_K = 384
_TAU = 1e-8


def _fused(x_ref, wq_ref, wk_ref, g_ref, y_ref):
    x = x_ref[0]
    s = jnp.mean(x, axis=1, keepdims=True)
    q = s * wq_ref[...]
    k = s * wk_ref[...]
    att = jax.lax.dot_general(
        q, k, (((1,), (1,)), ((), ())),
        preferred_element_type=jnp.float32) / math.sqrt(_C)
    att = jax.nn.softmax(att, axis=-1)
    scores = jnp.mean(att, axis=0, keepdims=True)
    noisy = scores + _TAU * g_ref[0]
    n_row = jnp.broadcast_to(noisy, (_C, _C))
    n_col = jnp.broadcast_to(noisy.reshape(_C, 1), (_C, _C))
    ii = jax.lax.broadcasted_iota(jnp.int32, (_C, _C), 0)
    jj = jax.lax.broadcasted_iota(jnp.int32, (_C, _C), 1)
    beats = (n_row > n_col) | ((n_row == n_col) & (jj < ii))
    rank = jnp.sum(beats.astype(jnp.int32), axis=1, keepdims=True)
    mask = (rank < _K).astype(jnp.float32)
    y_ref[0] = x * mask


def kernel(x, Wq, Wk):
    B, C, T = x.shape
    u = jax.random.uniform(jax.random.key(42), (B, C), minval=1e-20, maxval=1.0)
    g = (-jnp.log(-jnp.log(u))).reshape(B, 1, C)
    return pl.pallas_call(
        _fused,
        grid=(B,),
        in_specs=[
            pl.BlockSpec((1, _C, _T), lambda b: (b, 0, 0)),
            pl.BlockSpec((_C, _C), lambda b: (0, 0)),
            pl.BlockSpec((_C, _C), lambda b: (0, 0)),
            pl.BlockSpec((1, 1, _C), lambda b: (b, 0, 0)),
        ],
        out_specs=pl.BlockSpec((1, _C, _T), lambda b: (b, 0, 0)),
        out_shape=jax.ShapeDtypeStruct((B, C, T), jnp.float32),
        compiler_params=pltpu.CompilerParams(
            dimension_semantics=("arbitrary",),
        ),
    )(x, Wq, Wk, g)
